# Initial kernel scaffold; baseline (speedup 1.0000x reference)
import jax
import jax.numpy as jnp
from jax.experimental import pallas as pl


def kernel(graph, r_ij, W_a, W_v, W_d):
    raise NotImplementedError("write your pallas kernel here")


# speedup vs baseline: 18.8339x; 18.8339x over previous
import functools

import jax
import jax.numpy as jnp
from jax import lax
from jax.experimental import pallas as pl
from jax.experimental.pallas import tpu as pltpu
from jax.experimental.pallas import tpu_sc as plsc

N_NODES = 50000
N_EDGES = 800000
R0 = 5.0
N_RAD = 8

FW = 64
NF = 52
HALF = 25088
NS = 16
TROWS = HALF + 16
ZROWS = TROWS // NS

GRP = 128
CHG = 2
EPAD = 802816
EB = 2048
MB = 1024


def _feat_body(r_ref, outl_ref, outr_ref):
    r = r_ref[...]
    x = r[:, 0:1]
    y = r[:, 1:2]
    z = r[:, 2:3]
    sumsq = x * x + y * y + z * z
    x_sq = sumsq * (1.0 / R0)
    env = jnp.maximum(1.0 - x_sq, 0.0)
    t = jnp.sqrt(jnp.clip(x_sq, 0.0, 1.0))
    npi = jnp.pi * lax.broadcasted_iota(jnp.int32, (1, N_RAD), 1).astype(jnp.float32)
    rad = jnp.cos(npi * t) * env
    s = 17.0 / R0
    inv = lax.rsqrt(1.0 + (s * s) * sumsq)
    h = r * (s * inv)
    rr = jnp.concatenate([h[:, a:a + 1] * h for a in range(3)], axis=1)
    pieces = [rad]
    for c in range(N_RAD):
        pieces.append(rad[:, c:c + 1] * h)
    for c in range(N_RAD):
        pieces.append(rad[:, c:c + 1] * rr)
    zpad = jnp.zeros((EB, FW - NF), jnp.float32)
    feats = jnp.concatenate(pieces, axis=1)
    outl_ref[...] = jnp.concatenate([feats[:, :NF], zpad], axis=1)
    outr_ref[...] = jnp.concatenate([feats[:, NF:], zpad], axis=1)


def _featurize(r_pad):
    nb = EPAD // EB
    return pl.pallas_call(
        _feat_body,
        grid=(nb,),
        in_specs=[pl.BlockSpec((EB, 3), lambda i: (i, 0))],
        out_specs=[pl.BlockSpec((EB, FW), lambda i: (i, 0))] * 2,
        out_shape=[jax.ShapeDtypeStruct((EPAD, FW), jnp.float32)] * 2,
    )(r_pad)


_CHUNKS_PER_TEC = EPAD // (NS * CHG * GRP)


def _sc_scatter_body(graph_hbm, feats_hbm, zer_hbm, out_hbm, gbuf, lbuf, fbuf, table):
    c = lax.axis_index("c")
    s = lax.axis_index("s")
    base = c * HALF
    pltpu.sync_copy(zer_hbm, table.at[pl.ds(s * ZROWS, ZROWS)])
    plsc.subcore_barrier()

    def chunk_body(kc, _):
        e0 = (s * _CHUNKS_PER_TEC + kc) * (CHG * GRP)
        pltpu.sync_copy(graph_hbm.at[pl.ds(e0, CHG * GRP)], gbuf)
        pltpu.sync_copy(feats_hbm.at[pl.ds(e0, CHG * GRP)], fbuf)
        for jr in range(CHG):
            for u in range(GRP // 16):
                g = gbuf[pl.ds(jr * GRP + u * 16, 16)]
                loc = g - base
                ok = (loc >= 0) & (loc < HALF)
                lbuf[jr, pl.ds(u * 16, 16)] = jnp.where(ok, loc, HALF)
        for jr in range(CHG):
            pltpu.sync_copy(fbuf.at[pl.ds(jr * GRP, GRP)],
                            table.at[lbuf.at[jr]], add=True)
        return 0

    lax.fori_loop(0, _CHUNKS_PER_TEC, chunk_body, 0)
    plsc.subcore_barrier()
    orows = HALF // NS
    pltpu.sync_copy(table.at[pl.ds(s * orows, orows)],
                    out_hbm.at[pl.ds(base + s * orows, orows)])


@functools.cache
def _sc_scatter():
    mesh = plsc.VectorSubcoreMesh(
        core_axis_name="c", subcore_axis_name="s", num_cores=2, num_subcores=NS)
    return pl.kernel(
        _sc_scatter_body,
        out_type=jax.ShapeDtypeStruct((2 * HALF, FW), jnp.float32),
        mesh=mesh,
        scratch_types=[
            pltpu.VMEM((CHG * GRP,), jnp.int32),
            pltpu.VMEM((CHG, GRP), jnp.int32),
            pltpu.VMEM((CHG * GRP, FW), jnp.float32),
            pltpu.VMEM_SHARED((TROWS, FW), jnp.float32),
        ],
        compiler_params=pltpu.CompilerParams(use_tc_tiling_on_sc=False),
    )


def _mm_body(al_ref, ar_ref, wl_ref, wr_ref, o_ref):
    o_ref[...] = (
        jnp.dot(al_ref[...], wl_ref[...], preferred_element_type=jnp.float32)
        + jnp.dot(ar_ref[...], wr_ref[...], preferred_element_type=jnp.float32))


def _matmul(al, ar, wl, wr):
    nb = (2 * HALF) // MB
    return pl.pallas_call(
        _mm_body,
        grid=(nb,),
        in_specs=[
            pl.BlockSpec((MB, FW), lambda i: (i, 0)),
            pl.BlockSpec((MB, FW), lambda i: (i, 0)),
            pl.BlockSpec((FW, 832), lambda i: (0, 0)),
            pl.BlockSpec((FW, 832), lambda i: (0, 0)),
        ],
        out_specs=pl.BlockSpec((MB, 832), lambda i: (i, 0)),
        out_shape=jax.ShapeDtypeStruct((2 * HALF, 832), jnp.float32),
    )(al, ar, wl, wr)


def _build_wbig(W_a, W_v, W_d):
    I3 = jnp.eye(3, dtype=jnp.float32)
    I9 = jnp.eye(9, dtype=jnp.float32)
    wv = jnp.einsum('cd,ij->cidj', W_v, I3).reshape(24, 192)
    wd = jnp.einsum('cd,ij->cidj', W_d, I9).reshape(72, 576)
    wbig = jnp.zeros((104, 832), jnp.float32)
    wbig = wbig.at[0:8, 0:64].set(W_a)
    wbig = wbig.at[8:32, 64:256].set(wv)
    wbig = wbig.at[32:104, 256:832].set(wd)
    return wbig


def kernel(graph, r_ij, W_a, W_v, W_d):
    npad = EPAD - N_EDGES
    graph_p = jnp.concatenate(
        [graph.astype(jnp.int32), jnp.full((npad,), 2 * HALF, jnp.int32)])
    r_p = jnp.concatenate([r_ij, jnp.zeros((npad, 3), jnp.float32)])
    fl, fr = _featurize(r_p)
    zer = jnp.zeros((ZROWS, FW), jnp.float32)
    scat = _sc_scatter()
    al = scat(graph_p, fl, zer)
    ar = scat(graph_p, fr, zer)
    wbig = _build_wbig(W_a, W_v, W_d)
    wpad = jnp.zeros((FW - NF, 832), jnp.float32)
    wl = jnp.concatenate([wbig[:NF], wpad])
    wr = jnp.concatenate([wbig[NF:], wpad])
    b = _matmul(al, ar, wl, wr)
    b = b[:N_NODES]
    return (b[:, 0:64],
            b[:, 64:256].reshape(N_NODES, 64, 3),
            b[:, 256:832].reshape(N_NODES, 64, 3, 3))


# speedup vs baseline: 25.7286x; 1.3661x over previous
import functools

import jax
import jax.numpy as jnp
from jax import lax
from jax.experimental import pallas as pl
from jax.experimental.pallas import tpu as pltpu
from jax.experimental.pallas import tpu_sc as plsc

N_NODES = 50000
N_EDGES = 800000
R0 = 5.0
N_RAD = 8

FW = 64
NF = 52
HALF = 25088
NS = 16
TROWS = HALF + 16
ZROWS = TROWS // NS

GRP = 128
CH = 256
EPAD = 802816
MB = 1024

_CHUNKS_PER_TEC = EPAD // (NS * CH)


def _rsqrt16(v):
    i = plsc.bitcast(v, jnp.int32)
    y = plsc.bitcast(jnp.int32(0x5F3759DF) - lax.shift_right_logical(i, 1),
                     jnp.float32)
    for _ in range(3):
        y = y * (1.5 - 0.5 * v * y * y)
    return y


def _cospi16(t):
    z = jnp.pi * (t - 0.5)
    z2 = z * z
    s = z * (1.0 + z2 * (-1.0 / 6.0 + z2 * (1.0 / 120.0 + z2 * (
        -1.0 / 5040.0 + z2 * (1.0 / 362880.0)))))
    return -s


def _products(half):
    prods = []
    for F in range(NF * half, NF * half + NF):
        if F < 8:
            prods.append(('r', F, 0))
        elif F < 32:
            k = F - 8
            prods.append(('v', k // 3, k % 3))
        else:
            k = F - 32
            prods.append(('d', k // 9, k % 9))
    return prods


def _make_sc_body(half):
    prods = _products(half)

    def body(graph_hbm, r3_hbm, zer_hbm, out_hbm, gbuf, rbuf, lbuf, fbuf, table):
        c = lax.axis_index("c")
        s = lax.axis_index("s")
        base = c * HALF
        pltpu.sync_copy(zer_hbm, table.at[pl.ds(s * ZROWS, ZROWS)])
        pltpu.sync_copy(zer_hbm.at[pl.ds(0, CH)], fbuf)
        plsc.subcore_barrier()

        lane = lax.iota(jnp.int32, 16)
        lane3 = lane * 3

        def chunk_body(kc, _):
            e0 = (s * _CHUNKS_PER_TEC + kc) * CH
            pltpu.sync_copy(graph_hbm.at[pl.ds(e0, CH)], gbuf)
            pltpu.sync_copy(r3_hbm.at[pl.ds(e0 * 3, CH * 3)], rbuf)
            for jr in range(CH // GRP):
                def sub_body(u, _):
                    o16 = jr * GRP + u * 16
                    g = gbuf[pl.ds(o16, 16)]
                    loc = g - base
                    ok = (loc >= 0) & (loc < HALF)
                    lbuf[jr, pl.ds(u * 16, 16)] = jnp.where(ok, loc, HALF)
                    p = lane3 + o16 * 3
                    x = plsc.load_gather(rbuf, [p])
                    y = plsc.load_gather(rbuf, [p + 1])
                    z = plsc.load_gather(rbuf, [p + 2])
                    sumsq = x * x + y * y + z * z
                    x_sq = sumsq * (1.0 / R0)
                    env = jnp.maximum(1.0 - x_sq, 0.0)
                    xc = jnp.maximum(jnp.minimum(x_sq, 1.0), 1e-12)
                    t = xc * _rsqrt16(xc)
                    cp = _cospi16(t)
                    cheb = [jnp.full((16,), 1.0, jnp.float32), cp]
                    for _n in range(2, N_RAD):
                        cheb.append(2.0 * cp * cheb[-1] - cheb[-2])
                    rad = [env] + [cn * env for cn in cheb[1:]]
                    q = (17.0 / R0) * _rsqrt16(1.0 + (17.0 / R0) ** 2 * sumsq)
                    h = [x * q, y * q, z * q]
                    rr = [None] * 9
                    for a in range(3):
                        for b in range(a, 3):
                            rr[3 * a + b] = h[a] * h[b]
                            rr[3 * b + a] = rr[3 * a + b]
                    le = lane + o16
                    for col, (kind, rc, di) in enumerate(prods):
                        if kind == 'r':
                            val = rad[rc]
                        elif kind == 'v':
                            val = rad[rc] * h[di]
                        else:
                            val = rad[rc] * rr[di]
                        fcol = jnp.full((16,), col, jnp.int32)
                        plsc.store_scatter(fbuf, [le, fcol], val)
                    return 0
                lax.fori_loop(0, GRP // 16, sub_body, 0)
            for jr in range(CH // GRP):
                pltpu.sync_copy(fbuf.at[pl.ds(jr * GRP, GRP)],
                                table.at[lbuf.at[jr]], add=True)
            return 0

        lax.fori_loop(0, _CHUNKS_PER_TEC, chunk_body, 0)
        plsc.subcore_barrier()
        orows = HALF // NS
        pltpu.sync_copy(table.at[pl.ds(s * orows, orows)],
                        out_hbm.at[pl.ds(base + s * orows, orows)])

    return body


@functools.cache
def _sc_embed(half):
    mesh = plsc.VectorSubcoreMesh(
        core_axis_name="c", subcore_axis_name="s", num_cores=2, num_subcores=NS)
    return pl.kernel(
        _make_sc_body(half),
        out_type=jax.ShapeDtypeStruct((2 * HALF, FW), jnp.float32),
        mesh=mesh,
        scratch_types=[
            pltpu.VMEM((CH,), jnp.int32),
            pltpu.VMEM((CH * 3,), jnp.float32),
            pltpu.VMEM((CH // GRP, GRP), jnp.int32),
            pltpu.VMEM((CH, FW), jnp.float32),
            pltpu.VMEM_SHARED((TROWS, FW), jnp.float32),
        ],
        compiler_params=pltpu.CompilerParams(
            use_tc_tiling_on_sc=False, needs_layout_passes=False),
    )


def _mm_body(al_ref, ar_ref, wl_ref, wr_ref, o_ref):
    o_ref[...] = (
        jnp.dot(al_ref[...], wl_ref[...], preferred_element_type=jnp.float32)
        + jnp.dot(ar_ref[...], wr_ref[...], preferred_element_type=jnp.float32))


def _matmul(al, ar, wl, wr):
    nb = (2 * HALF) // MB
    return pl.pallas_call(
        _mm_body,
        grid=(nb,),
        in_specs=[
            pl.BlockSpec((MB, FW), lambda i: (i, 0)),
            pl.BlockSpec((MB, FW), lambda i: (i, 0)),
            pl.BlockSpec((FW, 832), lambda i: (0, 0)),
            pl.BlockSpec((FW, 832), lambda i: (0, 0)),
        ],
        out_specs=pl.BlockSpec((MB, 832), lambda i: (i, 0)),
        out_shape=jax.ShapeDtypeStruct((2 * HALF, 832), jnp.float32),
    )(al, ar, wl, wr)


def _build_wbig(W_a, W_v, W_d):
    I3 = jnp.eye(3, dtype=jnp.float32)
    I9 = jnp.eye(9, dtype=jnp.float32)
    wv = jnp.einsum('cd,ij->cidj', W_v, I3).reshape(24, 192)
    wd = jnp.einsum('cd,ij->cidj', W_d, I9).reshape(72, 576)
    wbig = jnp.zeros((104, 832), jnp.float32)
    wbig = wbig.at[0:8, 0:64].set(W_a)
    wbig = wbig.at[8:32, 64:256].set(wv)
    wbig = wbig.at[32:104, 256:832].set(wd)
    return wbig


def kernel(graph, r_ij, W_a, W_v, W_d):
    npad = EPAD - N_EDGES
    graph_p = jnp.concatenate(
        [graph.astype(jnp.int32), jnp.full((npad,), 2 * HALF, jnp.int32)])
    r3 = jnp.concatenate(
        [r_ij, jnp.zeros((npad, 3), jnp.float32)]).reshape(-1)
    zer = jnp.zeros((ZROWS, FW), jnp.float32)
    al = _sc_embed(0)(graph_p, r3, zer)
    ar = _sc_embed(1)(graph_p, r3, zer)
    wbig = _build_wbig(W_a, W_v, W_d)
    wpad = jnp.zeros((FW - NF, 832), jnp.float32)
    wl = jnp.concatenate([wbig[:NF], wpad])
    wr = jnp.concatenate([wbig[NF:], wpad])
    b = _matmul(al, ar, wl, wr)
    b = b[:N_NODES]
    return (b[:, 0:64],
            b[:, 64:256].reshape(N_NODES, 64, 3),
            b[:, 256:832].reshape(N_NODES, 64, 3, 3))
